# Initial kernel scaffold; baseline (speedup 1.0000x reference)
import jax
import jax.numpy as jnp
from jax.experimental import pallas as pl


def kernel(lf_fea, W1, W2):
    raise NotImplementedError("write your pallas kernel here")


# speedup vs baseline: 1.0266x; 1.0266x over previous
import functools

import jax
import jax.numpy as jnp
from jax.experimental import pallas as pl
from jax.experimental.pallas import tpu as pltpu

N = 1
AN = 5
C = 64
H = 64
W = 64
PSH = 8
K = 6
PNH = H // PSH
P = AN * H
D = C * AN * PSH
NB = N * PNH


def _dd_topk_body(xq_ref, lf_ref, idx_ref):
    xq = xq_ref[0]
    lf = lf_ref[0]
    g = jax.lax.dot_general(xq, lf, (((1,), (0,)), ((), ())),
                            preferred_element_type=jnp.float32)
    dd = -2.0 * g
    rown = jnp.sum(xq * xq, axis=1, keepdims=True)
    coln = jnp.sum(lf * lf, axis=0, keepdims=True)
    dd = dd + rown
    dd = dd + coln
    cols = jax.lax.broadcasted_iota(jnp.int32, (P, P), 1)
    work = dd
    big = jnp.float32(jnp.inf)
    for k in range(K):
        m = jnp.min(work, axis=1, keepdims=True)
        sel = jnp.where(work == m, cols, jnp.int32(2**30))
        j = jnp.min(sel, axis=1)
        idx_ref[0, :, k] = j
        work = jnp.where(cols == j[:, None], big, work)


@jax.jit
def _dd_topk(xq, lf):
    return pl.pallas_call(
        _dd_topk_body,
        grid=(NB,),
        in_specs=[
            pl.BlockSpec((1, P, D), lambda b: (b, 0, 0)),
            pl.BlockSpec((1, D, P), lambda b: (b, 0, 0)),
        ],
        out_specs=pl.BlockSpec((1, P, K), lambda b: (b, 0, 0)),
        out_shape=jax.ShapeDtypeStruct((NB, P, K), jnp.int32),
    )(xq, lf)


def _conv2d(x, w, padding):
    return jax.lax.conv_general_dilated(x, w, window_strides=(1, 1), padding=padding,
                                        dimension_numbers=('NCHW', 'OIHW', 'NCHW'))


def _lrelu(x):
    return jnp.where(x > 0, x, 0.1 * x)


def kernel(lf_fea, W1, W2):
    an2 = AN * AN
    x = lf_fea.reshape(N, AN, AN, C, H, W).transpose(0, 2, 1, 3, 5, 4).reshape(N * an2, C, W, H)
    t = x.reshape(N, AN, AN, C, W, H)
    lf_ver = t.transpose(0, 2, 5, 3, 1, 4).reshape(N * AN * H, C, AN, W)
    t2 = x.reshape(N, AN, AN, C, PNH, PSH, H)
    lf = t2.transpose(0, 4, 3, 1, 5, 2, 6).reshape(N * PNH, C * AN * PSH, AN * H)
    xq = jnp.transpose(lf, (0, 2, 1))

    idx = _dd_topk(xq, lf)

    idx_flat = idx.reshape(NB, 1, P * K)
    sel = jnp.take_along_axis(lf, idx_flat, axis=2)
    s = sel.reshape(N, PNH, C, AN, PSH, AN, H, K)
    s = s.transpose(0, 5, 6, 7, 2, 3, 1, 4).reshape(N * AN * H, K * C, AN, PNH * PSH)
    a1 = _lrelu(_conv2d(s, W1, 'VALID'))
    cat = jnp.concatenate([lf_ver, a1], axis=1)
    a2 = _lrelu(_conv2d(cat, W2, ((1, 1), (1, 1))))
    r = a2.reshape(N, AN, W, C, AN, W).transpose(0, 4, 1, 3, 5, 2).reshape(N * an2, C, W, W)
    out = r.reshape(N, AN, AN, C, W, W).transpose(0, 2, 1, 3, 5, 4).reshape(N * an2, C, W, W)
    return out
